# baseline (speedup 1.0000x reference)
import functools

import jax
import jax.numpy as jnp
from jax import lax
from jax.experimental import pallas as pl
from jax.experimental.pallas import tpu as pltpu
from jax.experimental.pallas import tpu_sc as plsc

B = 16384
H = 64
NC = 2
NS = 16
NW = NC * NS
BPW = B // NW
CH = 128
NCHUNK = BPW // CH


def _mf_body(user_hbm, recipe_hbm, uemb_hbm, remb_hbm, ubias_hbm, rbias_hbm,
             out_hbm, uidx_v, ridx_v, urows_v, rrows_v, ub_v, rb_v, out_v,
             m_v, sem):
    wid = lax.axis_index("c") * NS + lax.axis_index("s")
    base = wid * BPW

    pltpu.sync_copy(user_hbm.at[wid], uidx_v)
    pltpu.sync_copy(recipe_hbm.at[wid], ridx_v)

    copies = []
    for j in range(NCHUNK):
        copies.append(pltpu.async_copy(
            uemb_hbm.at[uidx_v.at[j]], urows_v.at[pl.ds(j * CH, CH)], sem))
        copies.append(pltpu.async_copy(
            remb_hbm.at[ridx_v.at[j]], rrows_v.at[pl.ds(j * CH, CH)], sem))
        copies.append(pltpu.async_copy(
            ubias_hbm.at[uidx_v.at[j]], ub_v.at[pl.ds(j * CH, CH)], sem))
        copies.append(pltpu.async_copy(
            rbias_hbm.at[ridx_v.at[j]], rb_v.at[pl.ds(j * CH, CH)], sem))
    for c in copies:
        c.wait()

    lanes = lax.iota(jnp.int32, 16)

    def group(g, _):
        eb = g * 16
        for i in range(16):
            e = eb + i
            acc = urows_v[e, pl.ds(0, 16)] * rrows_v[e, pl.ds(0, 16)]
            for k in range(1, H // 16):
                acc = acc + urows_v[e, pl.ds(k * 16, 16)] * rrows_v[e, pl.ds(k * 16, 16)]
            plsc.store_scatter(m_v, [lanes * 17 + i], acc)
        sv = m_v[pl.ds(0, 16)]
        for l in range(1, 16):
            sv = sv + m_v[pl.ds(l * 17, 16)]
        sv = sv + ub_v[pl.ds(eb, 16)] + rb_v[pl.ds(eb, 16)]
        out_v[pl.ds(eb, 16)] = sv
        return _

    lax.fori_loop(0, BPW // 16, group, None)

    pltpu.sync_copy(out_v, out_hbm.at[pl.ds(base, BPW)])


@jax.jit
def _mf_call(user, recipe, user_emb, recipe_emb, user_bias, recipe_bias):
    mesh = plsc.VectorSubcoreMesh(core_axis_name="c", subcore_axis_name="s")
    return pl.kernel(
        _mf_body,
        out_type=jax.ShapeDtypeStruct((B,), jnp.float32),
        mesh=mesh,
        compiler_params=pltpu.CompilerParams(
            needs_layout_passes=False, use_tc_tiling_on_sc=False),
        scratch_types=[
            pltpu.VMEM((NCHUNK, CH), jnp.int32),
            pltpu.VMEM((NCHUNK, CH), jnp.int32),
            pltpu.VMEM((BPW, H), jnp.float32),
            pltpu.VMEM((BPW, H), jnp.float32),
            pltpu.VMEM((BPW,), jnp.float32),
            pltpu.VMEM((BPW,), jnp.float32),
            pltpu.VMEM((BPW,), jnp.float32),
            pltpu.VMEM((16 * 17,), jnp.float32),
            pltpu.SemaphoreType.DMA,
        ],
    )(user, recipe, user_emb, recipe_emb, user_bias, recipe_bias)


def kernel(user, recipe, user_emb, recipe_emb, user_bias, recipe_bias):
    user = user.astype(jnp.int32).reshape(NW, NCHUNK, CH)
    recipe = recipe.astype(jnp.int32).reshape(NW, NCHUNK, CH)
    ub = user_bias.reshape(-1)
    rb = recipe_bias.reshape(-1)
    return _mf_call(user, recipe, user_emb, recipe_emb, ub, rb)
